# baseline (speedup 1.0000x reference)
import functools

import jax
import jax.numpy as jnp
from jax import lax
from jax.experimental import pallas as pl
from jax.experimental.pallas import tpu as pltpu
from jax.experimental.pallas import tpu_sc as plsc

_NUM_CORES = 2
_NUM_SUBCORES = 16
_NUM_WORKERS = _NUM_CORES * _NUM_SUBCORES


def _sc_gather(table, idx, batch, d_model):
    b_per_w = batch // _NUM_SUBCORES
    mesh = plsc.VectorSubcoreMesh(
        core_axis_name="c", subcore_axis_name="s", num_cores=1)

    @functools.partial(
        pl.kernel,
        mesh=mesh,
        out_type=jax.ShapeDtypeStruct((batch, d_model), jnp.float32),
        scratch_types=[
            pltpu.VMEM((2, b_per_w // 2), jnp.int32),
            pltpu.VMEM((b_per_w, d_model), jnp.float32),
            pltpu.SemaphoreType.DMA,
            pltpu.SemaphoreType.DMA,
        ],
    )
    def gather_kernel(table_hbm, idx_hbm, out_hbm, idx_v, rows_v, s0, s1):
        wid = lax.axis_index("s")
        base = wid * b_per_w
        h = b_per_w // 2
        pltpu.sync_copy(idx_hbm.at[pl.ds(base, h)], idx_v.at[0])
        pltpu.sync_copy(idx_hbm.at[pl.ds(base + h, h)], idx_v.at[1])
        a = pltpu.async_copy(
            table_hbm.at[idx_v.at[0]], rows_v.at[pl.ds(0, h)], s0)
        b = pltpu.async_copy(
            table_hbm.at[idx_v.at[1]], rows_v.at[pl.ds(h, h)], s1)
        a.wait()
        b.wait()
        pltpu.sync_copy(rows_v, out_hbm.at[pl.ds(base, b_per_w)])

    return gather_kernel(table, idx)


_FEATURES_PER_BLOCK = 5


def _tc_add_body(x_ref, ft_ref, te_ref, o_ref):
    o_ref[...] = x_ref[...] + ft_ref[...] + te_ref[...][None]


def kernel(x, time_indices, feature_type_encoding, time_encoding):
    num_features, batch, d_model = x.shape
    half = d_model // 2

    zf = jnp.zeros((num_features, half), jnp.float32)
    ft_i = jnp.stack([feature_type_encoding, zf], axis=-1).reshape(
        num_features, 1, d_model)
    zt = jnp.zeros((time_encoding.shape[0], half), jnp.float32)
    te_i = jnp.stack([zt, time_encoding], axis=-1).reshape(
        time_encoding.shape[0], d_model)

    te_full = _sc_gather(te_i, time_indices, batch, d_model)

    fb = _FEATURES_PER_BLOCK
    out = pl.pallas_call(
        _tc_add_body,
        grid=(num_features // fb,),
        in_specs=[
            pl.BlockSpec((fb, batch, d_model), lambda f: (f, 0, 0)),
            pl.BlockSpec((fb, 1, d_model), lambda f: (f, 0, 0)),
            pl.BlockSpec((batch, d_model), lambda f: (0, 0)),
        ],
        out_specs=pl.BlockSpec((fb, batch, d_model), lambda f: (f, 0, 0)),
        out_shape=jax.ShapeDtypeStruct(x.shape, x.dtype),
        compiler_params=pltpu.CompilerParams(
            dimension_semantics=("parallel",),
        ),
    )(x, ft_i, te_full)
    return out
